# baseline (speedup 1.0000x reference)
import functools
import jax
import jax.numpy as jnp
from jax.experimental import pallas as pl
from jax.experimental.pallas import tpu as pltpu

_TB = 512


def _fused_body(k, nb, x_ref, w_ref, o_ref, scores_ref, xring_ref, srow_ref,
                thr_ref):
    r = pl.program_id(0)
    j = pl.program_id(1)
    nj = pl.num_programs(1)
    tb = o_ref.shape[1]
    nring = xring_ref.shape[0]

    @pl.when(r < nb)
    def _():
        xb = x_ref[0]
        s_col = jnp.dot(xb, w_ref[...])
        scores_ref[...] = s_col.reshape(1, 1, tb)
        slot = (r * nj + j) % nring
        xring_ref[pl.ds(slot, 1)] = xb[None]
        srow_ref[pl.ds(r % 2, 1), 0, pl.ds(j * tb, tb)] = (
            s_col.reshape(1, tb))

        @pl.when(j == nj - 1)
        def _():
            s = srow_ref[pl.ds(r % 2, 1)]
            sb = jax.lax.bitcast_convert_type(s, jnp.int32)
            key = sb ^ ((sb >> 31) & jnp.int32(0x7FFFFFFF))
            mint = jnp.int32(-2147483648)

            def step(i, t_u):
                bit = jnp.left_shift(jnp.int32(1), 31 - i)
                cand = t_u | bit
                cnt = jnp.sum((key >= (cand ^ mint)).astype(jnp.int32))
                return jnp.where(cnt >= k, cand, t_u)

            t_u = jax.lax.fori_loop(0, 32, step, jnp.int32(0))
            t_s = t_u ^ mint
            fbits = jnp.where(t_s >= 0, t_s, t_s ^ jnp.int32(0x7FFFFFFF))
            thr_ref[r] = jax.lax.bitcast_convert_type(fbits, jnp.float32)

    @pl.when(r >= 1)
    def _():
        slot_c = ((r - 1) * nj + j) % nring
        xb1 = xring_ref[pl.ds(slot_c, 1)][0]
        s1 = srow_ref[pl.ds((r - 1) % 2, 1), 0, pl.ds(j * tb, tb)]
        s_col1 = s1.reshape(tb, 1)
        thr = thr_ref[r - 1]
        o_ref[0] = jnp.where(s_col1 >= thr, jax.nn.gelu(xb1), xb1)


def kernel(x, W_gate):
    B, T, D = x.shape
    k = max(1, int(T * 0.5))
    w = W_gate.reshape(D, 1)
    nj = T // _TB

    out, scores3d = pl.pallas_call(
        functools.partial(_fused_body, k, B),
        grid=(B + 1, nj),
        in_specs=[
            pl.BlockSpec(
                (1, _TB, D),
                lambda r, j: (jnp.minimum(r, B - 1),
                              jnp.where(r < B, j, nj - 1), 0)),
            pl.BlockSpec((D, 1), lambda r, j: (0, 0)),
        ],
        out_specs=[
            pl.BlockSpec(
                (1, _TB, D),
                lambda r, j: (jnp.maximum(r - 1, 0),
                              jnp.where(r >= 1, j, 0), 0)),
            pl.BlockSpec(
                (1, 1, _TB),
                lambda r, j: (jnp.minimum(r, B - 1), 0,
                              jnp.where(r < B, j, nj - 1))),
        ],
        out_shape=[
            jax.ShapeDtypeStruct((B, T, D), jnp.float32),
            jax.ShapeDtypeStruct((B, 1, T), jnp.float32),
        ],
        scratch_shapes=[
            pltpu.VMEM((nj + 1, _TB, D), jnp.float32),
            pltpu.VMEM((2, 1, T), jnp.float32),
            pltpu.SMEM((B,), jnp.float32),
        ],
        compiler_params=pltpu.CompilerParams(
            vmem_limit_bytes=100 * 1024 * 1024,
        ),
    )(x, w)

    return (out, scores3d.reshape(B, T))


# speedup vs baseline: 1.2824x; 1.2824x over previous
import functools
import jax
import jax.numpy as jnp
from jax.experimental import pallas as pl
from jax.experimental.pallas import tpu as pltpu

_TB = 1024


def _fused_body(k, nb, x_ref, w_ref, o_ref, scores_ref, xring_ref, srow_ref,
                thr_ref):
    r = pl.program_id(0)
    j = pl.program_id(1)
    nj = pl.num_programs(1)
    tb = o_ref.shape[1]
    nring = xring_ref.shape[0]

    @pl.when(r < nb)
    def _():
        xb = x_ref[0]
        s_col = jnp.dot(xb, w_ref[...])
        scores_ref[...] = s_col.reshape(1, 1, tb)
        slot = (r * nj + j) % nring
        xring_ref[pl.ds(slot, 1)] = xb[None]
        srow_ref[pl.ds(r % 2, 1), 0, pl.ds(j * tb, tb)] = (
            s_col.reshape(1, tb))

        @pl.when(j == nj - 1)
        def _():
            t = srow_ref.shape[2]
            s = srow_ref[pl.ds(r % 2, 1)].reshape(8, t // 8)
            sb = jax.lax.bitcast_convert_type(s, jnp.int32)
            key = sb ^ ((sb >> 31) & jnp.int32(0x7FFFFFFF))
            mint = jnp.int32(-2147483648)

            def step(i, t_u):
                bit = jnp.left_shift(jnp.int32(1), 31 - i)
                cand = t_u | bit
                cnt = jnp.sum((key >= (cand ^ mint)).astype(jnp.int32))
                return jnp.where(cnt >= k, cand, t_u)

            t_u = jax.lax.fori_loop(0, 32, step, jnp.int32(0))
            t_s = t_u ^ mint
            fbits = jnp.where(t_s >= 0, t_s, t_s ^ jnp.int32(0x7FFFFFFF))
            thr_ref[r] = jax.lax.bitcast_convert_type(fbits, jnp.float32)

    @pl.when(r >= 1)
    def _():
        slot_c = ((r - 1) * nj + j) % nring
        xb1 = xring_ref[pl.ds(slot_c, 1)][0]
        s1 = srow_ref[pl.ds((r - 1) % 2, 1), 0, pl.ds(j * tb, tb)]
        s_col1 = s1.reshape(tb, 1)
        thr = thr_ref[r - 1]
        o_ref[0] = jnp.where(s_col1 >= thr, jax.nn.gelu(xb1), xb1)


def kernel(x, W_gate):
    B, T, D = x.shape
    k = max(1, int(T * 0.5))
    w = W_gate.reshape(D, 1)
    nj = T // _TB

    out, scores3d = pl.pallas_call(
        functools.partial(_fused_body, k, B),
        grid=(B + 1, nj),
        in_specs=[
            pl.BlockSpec(
                (1, _TB, D),
                lambda r, j: (jnp.minimum(r, B - 1),
                              jnp.where(r < B, j, nj - 1), 0)),
            pl.BlockSpec((D, 1), lambda r, j: (0, 0)),
        ],
        out_specs=[
            pl.BlockSpec(
                (1, _TB, D),
                lambda r, j: (jnp.maximum(r - 1, 0),
                              jnp.where(r >= 1, j, 0), 0)),
            pl.BlockSpec(
                (1, 1, _TB),
                lambda r, j: (jnp.minimum(r, B - 1), 0,
                              jnp.where(r < B, j, nj - 1))),
        ],
        out_shape=[
            jax.ShapeDtypeStruct((B, T, D), jnp.float32),
            jax.ShapeDtypeStruct((B, 1, T), jnp.float32),
        ],
        scratch_shapes=[
            pltpu.VMEM((nj + 1, _TB, D), jnp.float32),
            pltpu.VMEM((2, 1, T), jnp.float32),
            pltpu.SMEM((B,), jnp.float32),
        ],
        compiler_params=pltpu.CompilerParams(
            vmem_limit_bytes=100 * 1024 * 1024,
        ),
    )(x, w)

    return (out, scores3d.reshape(B, T))
